# baseline (speedup 1.0000x reference)
import functools

import jax
import jax.numpy as jnp
from jax import lax
from jax.experimental import pallas as pl
from jax.experimental.pallas import tpu as pltpu
from jax.experimental.pallas import tpu_sc as plsc

G = 4096
N = 65536
V = 131072
F = 64
D_IN = 16
D_OUT = 8
PK = 8

NC = 2
NS = 16
NW = NC * NS
L = 16

_mesh = functools.partial(
    plsc.VectorSubcoreMesh, core_axis_name="c", subcore_axis_name="s"
)
_sc_params = pltpu.CompilerParams(use_tc_tiling_on_sc=False,
                                  needs_layout_passes=False)

_BLK = 1024


def _tables_body(xq_ref, xt_ref, g_ref, wbig_ref, bbig_ref, bilt_ref,
                 wsel_ref, w2_ref, bdec_ref, xq_out, xtc_out, s_ref):
    i = pl.program_id(0)
    xq_out[...] = jax.nn.relu(
        jnp.dot(xq_ref[...], wbig_ref[...], preferred_element_type=jnp.float32)
        + bbig_ref[...]
    )
    ct_big = jnp.dot(bilt_ref[...], wsel_ref[...],
                     preferred_element_type=jnp.float32)
    xt_lat = jax.nn.relu(
        jnp.dot(xt_ref[...], wbig_ref[...], preferred_element_type=jnp.float32)
        + bbig_ref[...]
    )
    xtc_out[...] = jnp.dot(xt_lat, ct_big, preferred_element_type=jnp.float32)

    @pl.when(i == 0)
    def _():
        s_ref[...] = (
            jnp.dot(g_ref[...], w2_ref[...], preferred_element_type=jnp.float32)
            + bdec_ref[...]
        )


def _tables(xqr, xtr, g_emb, wbig, bbig, bil_t_big, wsel_big, w2, bdec2):
    nblk = (N // PK) // _BLK
    return pl.pallas_call(
        _tables_body,
        grid=(nblk,),
        in_specs=[
            pl.BlockSpec((_BLK, PK * F), lambda i: (i, 0)),
            pl.BlockSpec((_BLK, PK * F), lambda i: (i, 0)),
            pl.BlockSpec((G, D_IN), lambda i: (0, 0)),
            pl.BlockSpec((PK * F, PK * D_IN), lambda i: (0, 0)),
            pl.BlockSpec((1, PK * D_IN), lambda i: (0, 0)),
            pl.BlockSpec((PK * D_IN, PK * D_IN * D_OUT), lambda i: (0, 0)),
            pl.BlockSpec((PK * D_IN * D_OUT, PK * D_IN), lambda i: (0, 0)),
            pl.BlockSpec((D_IN, 1), lambda i: (0, 0)),
            pl.BlockSpec((1, 1), lambda i: (0, 0)),
        ],
        out_specs=[
            pl.BlockSpec((_BLK, PK * D_IN), lambda i: (i, 0)),
            pl.BlockSpec((_BLK, PK * D_IN), lambda i: (i, 0)),
            pl.BlockSpec((G, 1), lambda i: (0, 0)),
        ],
        out_shape=[
            jax.ShapeDtypeStruct((N // PK, PK * D_IN), jnp.float32),
            jax.ShapeDtypeStruct((N // PK, PK * D_IN), jnp.float32),
            jax.ShapeDtypeStruct((G, 1), jnp.float32),
        ],
    )(xqr, xtr, g_emb, wbig, bbig, bil_t_big, wsel_big, w2, bdec2)


_UG_PER_W = G // NW


def _u_gather_body(xqp_hbm, uidx_hbm, out_hbm, idx_v, pidx_v, rows_v, out_v,
                   sem):
    wid = lax.axis_index("s") * NC + lax.axis_index("c")
    base = wid * _UG_PER_W
    pltpu.sync_copy(uidx_hbm.at[pl.ds(base, _UG_PER_W)], idx_v)
    for t in range(_UG_PER_W // L):
        u = idx_v[pl.ds(t * L, L)]
        pidx_v[pl.ds(t * L, L)] = u // PK
    pltpu.async_copy(xqp_hbm.at[pidx_v], rows_v, sem).wait()
    lane = jnp.arange(L, dtype=jnp.int32)
    for t in range(_UG_PER_W // L):
        u = idx_v[pl.ds(t * L, L)]
        lbase = (u % PK) * D_IN
        rows16 = lane + t * L
        for d in range(D_IN):
            val = plsc.load_gather(rows_v, [rows16, lbase + d])
            plsc.store_scatter(out_v, [rows16, jnp.full((L,), d, jnp.int32)],
                               val)
    pltpu.sync_copy(out_v, out_hbm.at[pl.ds(base, _UG_PER_W)])


_u_gather = pl.kernel(
    _u_gather_body,
    out_type=jax.ShapeDtypeStruct((G, D_IN), jnp.float32),
    mesh=_mesh(),
    compiler_params=_sc_params,
    scratch_types=[
        pltpu.VMEM((_UG_PER_W,), jnp.int32),
        pltpu.VMEM((_UG_PER_W,), jnp.int32),
        pltpu.VMEM((_UG_PER_W, PK * D_IN), jnp.float32),
        pltpu.VMEM((_UG_PER_W, D_IN), jnp.float32),
        pltpu.SemaphoreType.DMA,
    ],
)


_R_PER_W = V // NW
_SUB = 1024
_NSUB = _R_PER_W // _SUB


def _main_body(xtc_hbm, latq_hbm, s_hbm, vidx_hbm, vcor_hbm, out_hbm,
               s_v, idx_v, cor_v, rows_v, qrows_v, out_v, semr, semq):
    wid = lax.axis_index("s") * NC + lax.axis_index("c")
    base = wid * _R_PER_W
    pltpu.sync_copy(vidx_hbm.at[pl.ds(base, _R_PER_W)], idx_v)
    pltpu.sync_copy(vcor_hbm.at[pl.ds(base, _R_PER_W)], cor_v)

    def copies(j, p):
        boff = p * _SUB
        rc = pltpu.make_async_copy(
            xtc_hbm.at[idx_v.at[pl.ds(j * _SUB, _SUB)]],
            rows_v.at[pl.ds(boff, _SUB)], semr.at[p])
        qc = pltpu.make_async_copy(
            latq_hbm.at[cor_v.at[pl.ds(j * _SUB, _SUB)]],
            qrows_v.at[pl.ds(boff, _SUB)], semq.at[p])
        return rc, qc

    rc0, qc0 = copies(0, 0)
    rc0.start()
    qc0.start()
    pltpu.sync_copy(s_hbm, s_v)

    lane = jnp.arange(L, dtype=jnp.int32)

    def sub(j, carry):
        p = jnp.remainder(j, 2)

        @pl.when(j + 1 < _NSUB)
        def _():
            rc, qc = copies(j + 1, 1 - p)
            rc.start()
            qc.start()

        rc, qc = copies(j, p)
        rc.wait()
        qc.wait()

        off = j * _SUB
        boff = p * _SUB
        for t in range(_SUB // L):
            r0 = t * L
            g16 = cor_v[pl.ds(off + r0, L)]
            acc = plsc.load_gather(s_v, [g16])
            rows16 = lane + boff + r0
            for d in range(D_IN):
                d16 = jnp.full((L,), d, jnp.int32)
                xt_d = plsc.load_gather(rows_v, [rows16, d16])
                q_d = plsc.load_gather(qrows_v, [rows16, d16])
                acc = acc + xt_d * q_d
            out_v[pl.ds(off + r0, L)] = acc
        return carry

    lax.fori_loop(0, _NSUB, sub, 0)
    pltpu.sync_copy(out_v, out_hbm.at[pl.ds(base, _R_PER_W)])


_main = pl.kernel(
    _main_body,
    out_type=jax.ShapeDtypeStruct((V,), jnp.float32),
    mesh=_mesh(),
    compiler_params=_sc_params,
    scratch_types=[
        pltpu.VMEM((G,), jnp.float32),
        pltpu.VMEM((_R_PER_W,), jnp.int32),
        pltpu.VMEM((_R_PER_W,), jnp.int32),
        pltpu.VMEM((2 * _SUB, D_IN), jnp.float32),
        pltpu.VMEM((2 * _SUB, D_IN), jnp.float32),
        pltpu.VMEM((_R_PER_W,), jnp.float32),
        pltpu.SemaphoreType.DMA((2,)),
        pltpu.SemaphoreType.DMA((2,)),
    ],
)


def kernel(x_q, x_t, u_idx, v_idx, v_corres, g_emb, W_enc, b_enc,
           bilinear_mat, W_dec, b_dec):
    eye8 = jnp.eye(PK, dtype=jnp.float32)
    wbig = jnp.kron(eye8, W_enc)
    bbig = jnp.tile(b_enc, PK).reshape(1, PK * D_IN)

    bil_t = jnp.transpose(bilinear_mat, (1, 0, 2)).reshape(D_IN, D_IN * D_OUT)
    wsel = jnp.kron(jnp.eye(D_IN, dtype=jnp.float32), W_dec[:D_OUT])
    bil_t_big = jnp.kron(eye8, bil_t)
    wsel_big = jnp.kron(eye8, wsel)
    w2 = W_dec[D_OUT:]
    bdec2 = b_dec.reshape(1, 1)

    xqr = x_q.reshape(N // PK, PK * F)
    xtr = x_t.reshape(N // PK, PK * F)
    xq_p, xtc_p, s = _tables(xqr, xtr, g_emb, wbig, bbig, bil_t_big,
                             wsel_big, w2, bdec2)
    lat_q = _u_gather(xq_p, u_idx)
    return _main(xtc_p.reshape(N, D_IN), lat_q, s.reshape(G), v_idx, v_corres)


# speedup vs baseline: 1.0487x; 1.0487x over previous
import functools

import jax
import jax.numpy as jnp
from jax import lax
from jax.experimental import pallas as pl
from jax.experimental.pallas import tpu as pltpu
from jax.experimental.pallas import tpu_sc as plsc

G = 4096
N = 65536
V = 131072
F = 64
D_IN = 16
D_OUT = 8
PK = 8

NC = 2
NS = 16
NW = NC * NS
L = 16

_mesh = functools.partial(
    plsc.VectorSubcoreMesh, core_axis_name="c", subcore_axis_name="s"
)
_sc_params = pltpu.CompilerParams(use_tc_tiling_on_sc=False,
                                  needs_layout_passes=False)

_BLK = 1024


def _tables_body(xq_ref, xt_ref, g_ref, wbig_ref, bbig_ref, bilt_ref,
                 wsel_ref, w2_ref, bdec_ref, xq_out, xtc_out, s_ref):
    i = pl.program_id(0)
    xq_out[...] = jax.nn.relu(
        jnp.dot(xq_ref[...], wbig_ref[...], preferred_element_type=jnp.float32)
        + bbig_ref[...]
    )
    ct_big = jnp.dot(bilt_ref[...], wsel_ref[...],
                     preferred_element_type=jnp.float32)
    xt_lat = jax.nn.relu(
        jnp.dot(xt_ref[...], wbig_ref[...], preferred_element_type=jnp.float32)
        + bbig_ref[...]
    )
    xtc_out[...] = jnp.dot(xt_lat, ct_big, preferred_element_type=jnp.float32)

    @pl.when(i == 0)
    def _():
        s_ref[...] = (
            jnp.dot(g_ref[...], w2_ref[...], preferred_element_type=jnp.float32)
            + bdec_ref[...]
        )


def _tables(xqr, xtr, g_emb, wbig, bbig, bil_t_big, wsel_big, w2, bdec2):
    nblk = (N // PK) // _BLK
    return pl.pallas_call(
        _tables_body,
        grid=(nblk,),
        in_specs=[
            pl.BlockSpec((_BLK, PK * F), lambda i: (i, 0)),
            pl.BlockSpec((_BLK, PK * F), lambda i: (i, 0)),
            pl.BlockSpec((G, D_IN), lambda i: (0, 0)),
            pl.BlockSpec((PK * F, PK * D_IN), lambda i: (0, 0)),
            pl.BlockSpec((1, PK * D_IN), lambda i: (0, 0)),
            pl.BlockSpec((PK * D_IN, PK * D_IN * D_OUT), lambda i: (0, 0)),
            pl.BlockSpec((PK * D_IN * D_OUT, PK * D_IN), lambda i: (0, 0)),
            pl.BlockSpec((D_IN, 1), lambda i: (0, 0)),
            pl.BlockSpec((1, 1), lambda i: (0, 0)),
        ],
        out_specs=[
            pl.BlockSpec((_BLK, PK * D_IN), lambda i: (i, 0)),
            pl.BlockSpec((_BLK, PK * D_IN), lambda i: (i, 0)),
            pl.BlockSpec((G, 1), lambda i: (0, 0)),
        ],
        out_shape=[
            jax.ShapeDtypeStruct((N // PK, PK * D_IN), jnp.float32),
            jax.ShapeDtypeStruct((N // PK, PK * D_IN), jnp.float32),
            jax.ShapeDtypeStruct((G, 1), jnp.float32),
        ],
    )(xqr, xtr, g_emb, wbig, bbig, bil_t_big, wsel_big, w2, bdec2)


_UG_PER_W = G // NW


def _u_gather_body(xqp_hbm, uidx_hbm, out_hbm, idx_v, pidx_v, rows_v, out_v,
                   sem):
    wid = lax.axis_index("s") * NC + lax.axis_index("c")
    base = wid * _UG_PER_W
    pltpu.sync_copy(uidx_hbm.at[pl.ds(base, _UG_PER_W)], idx_v)
    for t in range(_UG_PER_W // L):
        u = idx_v[pl.ds(t * L, L)]
        pidx_v[pl.ds(t * L, L)] = u // PK
    pltpu.async_copy(xqp_hbm.at[pidx_v], rows_v, sem).wait()
    lane = jnp.arange(L, dtype=jnp.int32)
    for t in range(_UG_PER_W // L):
        u = idx_v[pl.ds(t * L, L)]
        lbase = (u % PK) * D_IN
        rows16 = lane + t * L
        for d in range(D_IN):
            val = plsc.load_gather(rows_v, [rows16, lbase + d])
            plsc.store_scatter(out_v, [rows16, jnp.full((L,), d, jnp.int32)],
                               val)
    pltpu.sync_copy(out_v, out_hbm.at[pl.ds(base, _UG_PER_W)])


_u_gather = pl.kernel(
    _u_gather_body,
    out_type=jax.ShapeDtypeStruct((G, D_IN), jnp.float32),
    mesh=_mesh(),
    compiler_params=_sc_params,
    scratch_types=[
        pltpu.VMEM((_UG_PER_W,), jnp.int32),
        pltpu.VMEM((_UG_PER_W,), jnp.int32),
        pltpu.VMEM((_UG_PER_W, PK * D_IN), jnp.float32),
        pltpu.VMEM((_UG_PER_W, D_IN), jnp.float32),
        pltpu.SemaphoreType.DMA,
    ],
)


_R_PER_W = V // NW
_SUB = 512
_NSUB = _R_PER_W // _SUB


def _main_body(xtc_hbm, latq_hbm, s_hbm, vidx_hbm, vcor_hbm, out_hbm,
               s_v, idx_v, cor_v, rows_v, qrows_v, out_v, semr, semq):
    wid = lax.axis_index("s") * NC + lax.axis_index("c")
    base = wid * _R_PER_W
    pltpu.sync_copy(vidx_hbm.at[pl.ds(base, _R_PER_W)], idx_v)
    pltpu.sync_copy(vcor_hbm.at[pl.ds(base, _R_PER_W)], cor_v)

    def copies(j, p):
        boff = p * _SUB
        rc = pltpu.make_async_copy(
            xtc_hbm.at[idx_v.at[pl.ds(j * _SUB, _SUB)]],
            rows_v.at[pl.ds(boff, _SUB)], semr.at[p])
        qc = pltpu.make_async_copy(
            latq_hbm.at[cor_v.at[pl.ds(j * _SUB, _SUB)]],
            qrows_v.at[pl.ds(boff, _SUB)], semq.at[p])
        return rc, qc

    rc0, qc0 = copies(0, 0)
    rc0.start()
    qc0.start()
    pltpu.sync_copy(s_hbm, s_v)

    lane = jnp.arange(L, dtype=jnp.int32)

    def sub(j, carry):
        p = jnp.remainder(j, 2)

        @pl.when(j + 1 < _NSUB)
        def _():
            rc, qc = copies(j + 1, 1 - p)
            rc.start()
            qc.start()

        rc, qc = copies(j, p)
        rc.wait()
        qc.wait()

        off = j * _SUB
        boff = p * _SUB
        for t in range(_SUB // L):
            r0 = t * L
            g16 = cor_v[pl.ds(off + r0, L)]
            acc = plsc.load_gather(s_v, [g16])
            rows16 = lane + boff + r0
            for d in range(D_IN):
                d16 = jnp.full((L,), d, jnp.int32)
                xt_d = plsc.load_gather(rows_v, [rows16, d16])
                q_d = plsc.load_gather(qrows_v, [rows16, d16])
                acc = acc + xt_d * q_d
            out_v[pl.ds(off + r0, L)] = acc
        return carry

    lax.fori_loop(0, _NSUB, sub, 0)
    pltpu.sync_copy(out_v, out_hbm.at[pl.ds(base, _R_PER_W)])


_main = pl.kernel(
    _main_body,
    out_type=jax.ShapeDtypeStruct((V,), jnp.float32),
    mesh=_mesh(),
    compiler_params=_sc_params,
    scratch_types=[
        pltpu.VMEM((G,), jnp.float32),
        pltpu.VMEM((_R_PER_W,), jnp.int32),
        pltpu.VMEM((_R_PER_W,), jnp.int32),
        pltpu.VMEM((2 * _SUB, D_IN), jnp.float32),
        pltpu.VMEM((2 * _SUB, D_IN), jnp.float32),
        pltpu.VMEM((_R_PER_W,), jnp.float32),
        pltpu.SemaphoreType.DMA((2,)),
        pltpu.SemaphoreType.DMA((2,)),
    ],
)


def kernel(x_q, x_t, u_idx, v_idx, v_corres, g_emb, W_enc, b_enc,
           bilinear_mat, W_dec, b_dec):
    eye8 = jnp.eye(PK, dtype=jnp.float32)
    wbig = jnp.kron(eye8, W_enc)
    bbig = jnp.tile(b_enc, PK).reshape(1, PK * D_IN)

    bil_t = jnp.transpose(bilinear_mat, (1, 0, 2)).reshape(D_IN, D_IN * D_OUT)
    wsel = jnp.kron(jnp.eye(D_IN, dtype=jnp.float32), W_dec[:D_OUT])
    bil_t_big = jnp.kron(eye8, bil_t)
    wsel_big = jnp.kron(eye8, wsel)
    w2 = W_dec[D_OUT:]
    bdec2 = b_dec.reshape(1, 1)

    xqr = x_q.reshape(N // PK, PK * F)
    xtr = x_t.reshape(N // PK, PK * F)
    xq_p, xtc_p, s = _tables(xqr, xtr, g_emb, wbig, bbig, bil_t_big,
                             wsel_big, w2, bdec2)
    lat_q = _u_gather(xq_p, u_idx)
    return _main(xtc_p.reshape(N, D_IN), lat_q, s.reshape(G), v_idx, v_corres)
